# baseline (speedup 1.0000x reference)
import functools

import jax
import jax.numpy as jnp
from jax import lax
from jax.experimental import pallas as pl
from jax.experimental.pallas import tpu as pltpu
from jax.experimental.pallas import tpu_sc as plsc

_B, _T, _D, _L = 16, 512, 512, 1024
_NC, _NS = 2, 16
_CH = 64
_Z0 = _B * (_T + 1)


def _ln(h, g, b):
    mu = jnp.mean(h, axis=-1, keepdims=True)
    var = jnp.mean((h - mu) * (h - mu), axis=-1, keepdims=True)
    return (h - mu) * lax.rsqrt(var + 1e-5) * g + b


def _front_body(x_ref, spk_ref, pt_ref,
                wd1_ref, bd1_ref, gd1_ref, bed1_ref,
                wd2_ref, bd2_ref, gd2_ref, bed2_ref, wdl_ref,
                wp1_ref, bp1_ref, gp1_ref, bep1_ref,
                wp2_ref, bp2_ref, gp2_ref, bep2_ref, wpl_ref,
                wpe_ref, bpe_ref,
                ld_ref, pp_ref, pe_ref):
    bi = pl.program_id(0)

    @pl.when(bi == _B)
    def _():
        pe_ref[0] = jnp.zeros((_T + 1, 2 * _D), jnp.float32)

    @pl.when(bi < _B)
    def _():
        _front_compute(x_ref, spk_ref, pt_ref,
                       wd1_ref, bd1_ref, gd1_ref, bed1_ref,
                       wd2_ref, bd2_ref, gd2_ref, bed2_ref, wdl_ref,
                       wp1_ref, bp1_ref, gp1_ref, bep1_ref,
                       wp2_ref, bp2_ref, gp2_ref, bep2_ref, wpl_ref,
                       wpe_ref, bpe_ref, ld_ref, pp_ref, pe_ref)


def _front_compute(x_ref, spk_ref, pt_ref,
                   wd1_ref, bd1_ref, gd1_ref, bed1_ref,
                   wd2_ref, bd2_ref, gd2_ref, bed2_ref, wdl_ref,
                   wp1_ref, bp1_ref, gp1_ref, bep1_ref,
                   wp2_ref, bp2_ref, gp2_ref, bep2_ref, wpl_ref,
                   wpe_ref, bpe_ref, ld_ref, pp_ref, pe_ref):
    xb = x_ref[0]
    zrow = jnp.zeros((1, _D), jnp.float32)

    def shifts(m):
        return (jnp.concatenate([zrow, m[:-1]], 0),
                jnp.concatenate([m[1:], zrow], 0))

    xm, xp = shifts(xb)

    def var_pred(w1, b1, g1, be1, w2, b2, g2, be2, wl):
        h = xm @ w1[0] + xb @ w1[1] + xp @ w1[2] + b1
        h = _ln(jnp.maximum(h, 0.0), g1, be1)
        hm, hp = shifts(h)
        h2 = hm @ w2[0] + h @ w2[1] + hp @ w2[2] + b2
        h2 = _ln(jnp.maximum(h2, 0.0), g2, be2)
        return jnp.sum(h2 * wl, axis=-1)

    ld_ref[0, 0, :] = var_pred(wd1_ref[...], bd1_ref[...], gd1_ref[...],
                               bed1_ref[...], wd2_ref[...], bd2_ref[...],
                               gd2_ref[...], bed2_ref[...], wdl_ref[...])
    pp_ref[0, 0, :] = var_pred(wp1_ref[...], bp1_ref[...], gp1_ref[...],
                               bep1_ref[...], wp2_ref[...], bp2_ref[...],
                               gp2_ref[...], bep2_ref[...], wpl_ref[...])

    pt = pt_ref[0]
    wpe = wpe_ref[...]
    acc = pt * wpe[4]
    for k in range(9):
        d = k - 4
        if d == 0:
            continue
        if d < 0:
            sh = jnp.concatenate([jnp.zeros((-d, 1), jnp.float32), pt[:d]], 0)
        else:
            sh = jnp.concatenate([pt[d:], jnp.zeros((d, 1), jnp.float32)], 0)
        acc = acc + sh * wpe[k]
    pe_ref[0, :_T, :_D] = xb
    pe_ref[0, :_T, _D:] = acc + bpe_ref[...] + spk_ref[0]
    pe_ref[0, _T:, :] = jnp.zeros((1, 2 * _D), jnp.float32)


def _front(x, spk, pt3, *weights):
    clamp = lambda b: jnp.minimum(b, _B - 1)
    full = lambda a: pl.BlockSpec(a.shape, lambda b: (0,) * a.ndim)
    in_specs = [
        pl.BlockSpec((1, _T, _D), lambda b: (clamp(b), 0, 0)),
        pl.BlockSpec((1, _T, _D), lambda b: (clamp(b), 0, 0)),
        pl.BlockSpec((1, _T, 1), lambda b: (clamp(b), 0, 0)),
    ] + [full(w) for w in weights]
    return pl.pallas_call(
        _front_body,
        grid=(_B + 1,),
        in_specs=in_specs,
        out_specs=[
            pl.BlockSpec((1, 1, _T), lambda b: (clamp(b), 0, 0)),
            pl.BlockSpec((1, 1, _T), lambda b: (clamp(b), 0, 0)),
            pl.BlockSpec((1, _T + 1, 2 * _D), lambda b: (b, 0, 0)),
        ],
        out_shape=[
            jax.ShapeDtypeStruct((_B, 1, _T), jnp.float32),
            jax.ShapeDtypeStruct((_B, 1, _T), jnp.float32),
            jax.ShapeDtypeStruct((_B + 1, _T + 1, 2 * _D), jnp.float32),
        ],
        compiler_params=pltpu.CompilerParams(
            dimension_semantics=("arbitrary",)),
    )(x, spk, pt3, *weights)


def _lr_body(src_hbm, dur_hbm, comb_hbm,
             dur_v, idx_v, buf_v, gsem, os0, os1):
    wid = lax.axis_index("s") * _NC + lax.axis_index("c")
    b = wid // 2
    half = wid % 2

    pltpu.sync_copy(dur_hbm.at[b], dur_v)

    lane16 = lax.iota(jnp.int32, 16)
    for i in range(_L // 16):
        idx_v[pl.ds(i * 16, 16)] = _Z0 + (i % 4) * 16 + lane16

    base = b * (_T + 1)
    lane = lax.iota(jnp.int32, 16)
    gdn = lax.GatherDimensionNumbers(
        offset_dims=(), collapsed_slice_dims=(0,), start_index_map=(0,))

    def shift16(c, s):
        src = jnp.maximum(lane - s, 0)
        g = lax.gather(c, src[:, None], gdn, (1,),
                       mode=lax.GatherScatterMode.PROMISE_IN_BOUNDS)
        return jnp.where(lane >= s, g, 0)

    def scan_body(j, carry):
        v = dur_v[pl.ds(j * 16, 16)]
        cum = v
        for s in (1, 2, 4, 8):
            cum = cum + shift16(cum, s)
        pos = cum - v + carry
        val = base + j * 16 + lane
        for r in range(3):
            m = (v > r) & (pos + r < _L)
            plsc.store_scatter(idx_v, [pos + r], val, mask=m)
        return carry + cum[15]

    mel_len = lax.fori_loop(0, _T // 16, scan_body, jnp.int32(0))

    for c in range(512 // _CH):
        start = half * 512 + c * _CH
        orow = b * _L + start

        @pl.when(start < mel_len)
        def _(start=start, orow=orow):
            pltpu.async_copy(src_hbm.at[idx_v.at[pl.ds(start, _CH)]],
                             buf_v, gsem).wait()
            pltpu.async_copy(buf_v, comb_hbm.at[pl.ds(orow, _CH)],
                             os0).wait()

        @pl.when(start >= mel_len)
        def _(orow=orow):
            pltpu.async_copy(src_hbm.at[pl.ds(_Z0, _CH)],
                             comb_hbm.at[pl.ds(orow, _CH)], os1).wait()


def _length_regulate(srcflat, dur):
    return pl.kernel(
        _lr_body,
        out_type=jax.ShapeDtypeStruct((_B * _L, 2 * _D), jnp.float32),
        mesh=plsc.VectorSubcoreMesh(core_axis_name="c", subcore_axis_name="s",
                                    num_cores=_NC, num_subcores=_NS),
        compiler_params=pltpu.CompilerParams(needs_layout_passes=False),
        scratch_types=[
            pltpu.VMEM((_T,), jnp.int32),
            pltpu.VMEM((_L,), jnp.int32),
            pltpu.VMEM((_CH, 2 * _D), jnp.float32),
        ] + [pltpu.SemaphoreType.DMA] * 3,
    )(srcflat, dur)


def kernel(x, speaker_embedding, src_mask, mel_mask, max_len, pitch_target,
           duration_target,
           w_dc1, b_dc1, g_d1, be_d1, w_dc2, b_dc2, g_d2, be_d2, w_dlin, b_dlin,
           w_pc1, b_pc1, g_p1, be_p1, w_pc2, b_pc2, g_p2, be_p2, w_plin, b_plin,
           w_pe, b_pe):
    taps = lambda w: jnp.transpose(w, (2, 1, 0))
    ld, pp, pe = _front(
        x, speaker_embedding, pitch_target[:, :, None],
        taps(w_dc1), b_dc1, g_d1, be_d1, taps(w_dc2), b_dc2, g_d2, be_d2,
        w_dlin,
        taps(w_pc1), b_pc1, g_p1, be_p1, taps(w_pc2), b_pc2, g_p2, be_p2,
        w_plin,
        jnp.transpose(w_pe[:, 0, :]), b_pe)

    comb = _length_regulate(pe.reshape((_B + 1) * (_T + 1), 2 * _D),
                            duration_target)
    text_f, pitch_f = comb[:, :_D], comb[:, _D:]

    log_dur = jnp.where(src_mask, 0.0, ld[:, 0] + b_dlin)
    pitch_pred = jnp.where(src_mask, 0.0, pp[:, 0] + b_plin)
    mel_len = jnp.sum(duration_target, axis=1)
    return (text_f.reshape(_B, _L, _D), pitch_f.reshape(_B, _L, _D),
            pitch_pred, log_dur, duration_target, mel_len, mel_mask)


# speedup vs baseline: 2.5371x; 2.5371x over previous
import functools

import jax
import jax.numpy as jnp
from jax import lax
from jax.experimental import pallas as pl
from jax.experimental.pallas import tpu as pltpu
from jax.experimental.pallas import tpu_sc as plsc

_B, _T, _D, _L = 16, 512, 512, 1024
_NC, _NS = 2, 16
_CH = 64
_Z0 = _B * (_T + 1)


def _ln(h, g, b):
    mu = jnp.mean(h, axis=-1, keepdims=True)
    var = jnp.mean((h - mu) * (h - mu), axis=-1, keepdims=True)
    return (h - mu) * lax.rsqrt(var + 1e-5) * g + b


def _front_body(x_ref, spk_ref, pt_ref,
                wd1_ref, bd1_ref, gd1_ref, bed1_ref,
                wd2_ref, bd2_ref, gd2_ref, bed2_ref, wdl_ref,
                wp1_ref, bp1_ref, gp1_ref, bep1_ref,
                wp2_ref, bp2_ref, gp2_ref, bep2_ref, wpl_ref,
                wpe_ref, bpe_ref,
                ld_ref, pp_ref, pe_ref):
    bi = pl.program_id(0)

    @pl.when(bi == _B)
    def _():
        pe_ref[0] = jnp.zeros((_T + 1, 2 * _D), jnp.float32)

    @pl.when(bi < _B)
    def _():
        _front_compute(x_ref, spk_ref, pt_ref,
                       wd1_ref, bd1_ref, gd1_ref, bed1_ref,
                       wd2_ref, bd2_ref, gd2_ref, bed2_ref, wdl_ref,
                       wp1_ref, bp1_ref, gp1_ref, bep1_ref,
                       wp2_ref, bp2_ref, gp2_ref, bep2_ref, wpl_ref,
                       wpe_ref, bpe_ref, ld_ref, pp_ref, pe_ref)


def _front_compute(x_ref, spk_ref, pt_ref,
                   wd1_ref, bd1_ref, gd1_ref, bed1_ref,
                   wd2_ref, bd2_ref, gd2_ref, bed2_ref, wdl_ref,
                   wp1_ref, bp1_ref, gp1_ref, bep1_ref,
                   wp2_ref, bp2_ref, gp2_ref, bep2_ref, wpl_ref,
                   wpe_ref, bpe_ref, ld_ref, pp_ref, pe_ref):
    xb = x_ref[0]
    zrow = jnp.zeros((1, _D), jnp.float32)

    def shifts(m):
        return (jnp.concatenate([zrow, m[:-1]], 0),
                jnp.concatenate([m[1:], zrow], 0))

    xm, xp = shifts(xb)

    def var_pred(w1, b1, g1, be1, w2, b2, g2, be2, wl):
        h = xm @ w1[0] + xb @ w1[1] + xp @ w1[2] + b1
        h = _ln(jnp.maximum(h, 0.0), g1, be1)
        hm, hp = shifts(h)
        h2 = hm @ w2[0] + h @ w2[1] + hp @ w2[2] + b2
        h2 = _ln(jnp.maximum(h2, 0.0), g2, be2)
        return jnp.sum(h2 * wl, axis=-1)

    ld_ref[0, 0, :] = var_pred(wd1_ref[...], bd1_ref[...], gd1_ref[...],
                               bed1_ref[...], wd2_ref[...], bd2_ref[...],
                               gd2_ref[...], bed2_ref[...], wdl_ref[...])
    pp_ref[0, 0, :] = var_pred(wp1_ref[...], bp1_ref[...], gp1_ref[...],
                               bep1_ref[...], wp2_ref[...], bp2_ref[...],
                               gp2_ref[...], bep2_ref[...], wpl_ref[...])

    pt = pt_ref[0]
    wpe = wpe_ref[...]
    acc = pt * wpe[4]
    for k in range(9):
        d = k - 4
        if d == 0:
            continue
        if d < 0:
            sh = jnp.concatenate([jnp.zeros((-d, 1), jnp.float32), pt[:d]], 0)
        else:
            sh = jnp.concatenate([pt[d:], jnp.zeros((d, 1), jnp.float32)], 0)
        acc = acc + sh * wpe[k]
    pe_ref[0, :_T, :_D] = xb
    pe_ref[0, :_T, _D:] = acc + bpe_ref[...] + spk_ref[0]
    pe_ref[0, _T:, :] = jnp.zeros((1, 2 * _D), jnp.float32)


def _front(x, spk, pt3, *weights):
    clamp = lambda b: jnp.minimum(b, _B - 1)
    full = lambda a: pl.BlockSpec(a.shape, lambda b: (0,) * a.ndim)
    in_specs = [
        pl.BlockSpec((1, _T, _D), lambda b: (clamp(b), 0, 0)),
        pl.BlockSpec((1, _T, _D), lambda b: (clamp(b), 0, 0)),
        pl.BlockSpec((1, _T, 1), lambda b: (clamp(b), 0, 0)),
    ] + [full(w) for w in weights]
    return pl.pallas_call(
        _front_body,
        grid=(_B + 1,),
        in_specs=in_specs,
        out_specs=[
            pl.BlockSpec((1, 1, _T), lambda b: (clamp(b), 0, 0)),
            pl.BlockSpec((1, 1, _T), lambda b: (clamp(b), 0, 0)),
            pl.BlockSpec((1, _T + 1, 2 * _D), lambda b: (b, 0, 0)),
        ],
        out_shape=[
            jax.ShapeDtypeStruct((_B, 1, _T), jnp.float32),
            jax.ShapeDtypeStruct((_B, 1, _T), jnp.float32),
            jax.ShapeDtypeStruct((_B + 1, _T + 1, 2 * _D), jnp.float32),
        ],
        compiler_params=pltpu.CompilerParams(
            dimension_semantics=("arbitrary",)),
    )(x, spk, pt3, *weights)


def _lr_body(src_hbm, dur_hbm, comb_hbm,
             dur_v, idx_v, buf_v, gsem, os0, os1):
    core = lax.axis_index("c")
    sub = lax.axis_index("s")
    b = core * 8 + sub // 2
    half = sub % 2

    pltpu.sync_copy(dur_hbm.at[b], dur_v)

    lane16 = lax.iota(jnp.int32, 16)
    for i in range(_L // 16):
        idx_v[pl.ds(i * 16, 16)] = _Z0 + (i % 4) * 16 + lane16

    base = b * (_T + 1)
    lane = lax.iota(jnp.int32, 16)
    gdn = lax.GatherDimensionNumbers(
        offset_dims=(), collapsed_slice_dims=(0,), start_index_map=(0,))

    def shift16(c, s):
        src = jnp.maximum(lane - s, 0)
        g = lax.gather(c, src[:, None], gdn, (1,),
                       mode=lax.GatherScatterMode.PROMISE_IN_BOUNDS)
        return jnp.where(lane >= s, g, 0)

    def scan_body(j, carry):
        v = dur_v[pl.ds(j * 16, 16)]
        cum = v
        for s in (1, 2, 4, 8):
            cum = cum + shift16(cum, s)
        pos = cum - v + carry
        val = base + j * 16 + lane
        for r in range(3):
            m = (v > r) & (pos + r < _L)
            plsc.store_scatter(idx_v, [pos + r], val, mask=m)
        return carry + cum[15]

    mel_len = lax.fori_loop(0, _T // 16, scan_body, jnp.int32(0))

    prev = None
    was_zero = jnp.bool_(False)
    for c in range(512 // _CH):
        start = half * 512 + c * _CH
        valid = start < mel_len
        if prev is not None:
            prev.wait()

        @pl.when(valid)
        def _(start=start):
            pltpu.async_copy(src_hbm.at[idx_v.at[pl.ds(start, _CH)]],
                             buf_v, gsem).wait()

        @pl.when(jnp.logical_not(valid | was_zero))
        def _():
            pltpu.async_copy(src_hbm.at[pl.ds(_Z0, _CH)], buf_v, gsem).wait()

        was_zero = jnp.logical_not(valid)
        prev = pltpu.async_copy(buf_v,
                                comb_hbm.at[pl.ds(b * _L + start, _CH)],
                                (os0, os1)[c % 2])
    prev.wait()


def _length_regulate(srcflat, dur):
    return pl.kernel(
        _lr_body,
        out_type=jax.ShapeDtypeStruct((_B * _L, 2 * _D), jnp.float32),
        mesh=plsc.VectorSubcoreMesh(core_axis_name="c", subcore_axis_name="s",
                                    num_cores=_NC, num_subcores=_NS),
        compiler_params=pltpu.CompilerParams(needs_layout_passes=False),
        scratch_types=[
            pltpu.VMEM((_T,), jnp.int32),
            pltpu.VMEM((_L,), jnp.int32),
            pltpu.VMEM((_CH, 2 * _D), jnp.float32),
        ] + [pltpu.SemaphoreType.DMA] * 3,
    )(srcflat, dur)


def kernel(x, speaker_embedding, src_mask, mel_mask, max_len, pitch_target,
           duration_target,
           w_dc1, b_dc1, g_d1, be_d1, w_dc2, b_dc2, g_d2, be_d2, w_dlin, b_dlin,
           w_pc1, b_pc1, g_p1, be_p1, w_pc2, b_pc2, g_p2, be_p2, w_plin, b_plin,
           w_pe, b_pe):
    taps = lambda w: jnp.transpose(w, (2, 1, 0))
    ld, pp, pe = _front(
        x, speaker_embedding, pitch_target[:, :, None],
        taps(w_dc1), b_dc1, g_d1, be_d1, taps(w_dc2), b_dc2, g_d2, be_d2,
        w_dlin,
        taps(w_pc1), b_pc1, g_p1, be_p1, taps(w_pc2), b_pc2, g_p2, be_p2,
        w_plin,
        jnp.transpose(w_pe[:, 0, :]), b_pe)

    comb = _length_regulate(pe.reshape((_B + 1) * (_T + 1), 2 * _D),
                            duration_target)
    text_f, pitch_f = comb[:, :_D], comb[:, _D:]

    log_dur = jnp.where(src_mask, 0.0, ld[:, 0] + b_dlin)
    pitch_pred = jnp.where(src_mask, 0.0, pp[:, 0] + b_plin)
    mel_len = jnp.sum(duration_target, axis=1)
    return (text_f.reshape(_B, _L, _D), pitch_f.reshape(_B, _L, _D),
            pitch_pred, log_dur, duration_target, mel_len, mel_mask)


# speedup vs baseline: 3.4902x; 1.3757x over previous
import functools

import jax
import jax.numpy as jnp
from jax import lax
from jax.experimental import pallas as pl
from jax.experimental.pallas import tpu as pltpu
from jax.experimental.pallas import tpu_sc as plsc

_B, _T, _D, _L = 16, 512, 512, 1024
_NC, _NS = 2, 16
_CH = 64
_Z0 = _B * _T


def _ln(h, g, b):
    mu = jnp.mean(h, axis=-1, keepdims=True)
    var = jnp.mean((h - mu) * (h - mu), axis=-1, keepdims=True)
    return (h - mu) * lax.rsqrt(var + 1e-5) * g + b


def _front_body(x_ref, spk_ref, pt_ref,
                wd1_ref, bd1_ref, gd1_ref, bed1_ref,
                wd2_ref, bd2_ref, gd2_ref, bed2_ref, wdl_ref,
                wp1_ref, bp1_ref, gp1_ref, bep1_ref,
                wp2_ref, bp2_ref, gp2_ref, bep2_ref, wpl_ref,
                wpe_ref, bpe_ref,
                ld_ref, pp_ref, pe_ref):
    bi = pl.program_id(0)

    @pl.when(bi == _B)
    def _():
        pe_ref[0] = jnp.zeros((_T, 2 * _D), jnp.float32)

    @pl.when(bi < _B)
    def _():
        _front_compute(x_ref, spk_ref, pt_ref,
                       wd1_ref, bd1_ref, gd1_ref, bed1_ref,
                       wd2_ref, bd2_ref, gd2_ref, bed2_ref, wdl_ref,
                       wp1_ref, bp1_ref, gp1_ref, bep1_ref,
                       wp2_ref, bp2_ref, gp2_ref, bep2_ref, wpl_ref,
                       wpe_ref, bpe_ref, ld_ref, pp_ref, pe_ref)


def _front_compute(x_ref, spk_ref, pt_ref,
                   wd1_ref, bd1_ref, gd1_ref, bed1_ref,
                   wd2_ref, bd2_ref, gd2_ref, bed2_ref, wdl_ref,
                   wp1_ref, bp1_ref, gp1_ref, bep1_ref,
                   wp2_ref, bp2_ref, gp2_ref, bep2_ref, wpl_ref,
                   wpe_ref, bpe_ref, ld_ref, pp_ref, pe_ref):
    xb = x_ref[0]
    zrow = jnp.zeros((1, _D), jnp.float32)

    def shifts(m):
        return (jnp.concatenate([zrow, m[:-1]], 0),
                jnp.concatenate([m[1:], zrow], 0))

    xm, xp = shifts(xb)

    def var_pred(w1, b1, g1, be1, w2, b2, g2, be2, wl):
        h = xm @ w1[0] + xb @ w1[1] + xp @ w1[2] + b1
        h = _ln(jnp.maximum(h, 0.0), g1, be1)
        hm, hp = shifts(h)
        h2 = hm @ w2[0] + h @ w2[1] + hp @ w2[2] + b2
        h2 = _ln(jnp.maximum(h2, 0.0), g2, be2)
        return jnp.sum(h2 * wl, axis=-1)

    ld_ref[0, 0, :] = var_pred(wd1_ref[...], bd1_ref[...], gd1_ref[...],
                               bed1_ref[...], wd2_ref[...], bd2_ref[...],
                               gd2_ref[...], bed2_ref[...], wdl_ref[...])
    pp_ref[0, 0, :] = var_pred(wp1_ref[...], bp1_ref[...], gp1_ref[...],
                               bep1_ref[...], wp2_ref[...], bp2_ref[...],
                               gp2_ref[...], bep2_ref[...], wpl_ref[...])

    pt = pt_ref[0]
    wpe = wpe_ref[...]
    acc = pt * wpe[4]
    for k in range(9):
        d = k - 4
        if d == 0:
            continue
        if d < 0:
            sh = jnp.concatenate([jnp.zeros((-d, 1), jnp.float32), pt[:d]], 0)
        else:
            sh = jnp.concatenate([pt[d:], jnp.zeros((d, 1), jnp.float32)], 0)
        acc = acc + sh * wpe[k]
    pe_ref[0, :, :_D] = xb
    pe_ref[0, :, _D:] = acc + bpe_ref[...] + spk_ref[0]


def _front(x, spk, pt3, *weights):
    clamp = lambda b: jnp.minimum(b, _B - 1)
    full = lambda a: pl.BlockSpec(a.shape, lambda b: (0,) * a.ndim)
    in_specs = [
        pl.BlockSpec((1, _T, _D), lambda b: (clamp(b), 0, 0)),
        pl.BlockSpec((1, _T, _D), lambda b: (clamp(b), 0, 0)),
        pl.BlockSpec((1, _T, 1), lambda b: (clamp(b), 0, 0)),
    ] + [full(w) for w in weights]
    return pl.pallas_call(
        _front_body,
        grid=(_B + 1,),
        in_specs=in_specs,
        out_specs=[
            pl.BlockSpec((1, 1, _T), lambda b: (clamp(b), 0, 0)),
            pl.BlockSpec((1, 1, _T), lambda b: (clamp(b), 0, 0)),
            pl.BlockSpec((1, _T, 2 * _D), lambda b: (b, 0, 0)),
        ],
        out_shape=[
            jax.ShapeDtypeStruct((_B, 1, _T), jnp.float32),
            jax.ShapeDtypeStruct((_B, 1, _T), jnp.float32),
            jax.ShapeDtypeStruct((_B + 1, _T, 2 * _D), jnp.float32),
        ],
        compiler_params=pltpu.CompilerParams(
            dimension_semantics=("arbitrary",)),
    )(x, spk, pt3, *weights)


def _lr_body(src_hbm, dur_hbm, text_hbm, pitch_hbm,
             dur_v, idx_v, buf_v, gsem, os0, os1):
    core = lax.axis_index("c")
    sub = lax.axis_index("s")
    b = core * 8 + sub // 2
    half = sub % 2

    pltpu.sync_copy(dur_hbm.at[b], dur_v)

    lane16 = lax.iota(jnp.int32, 16)
    for i in range(_L // 16):
        idx_v[pl.ds(i * 16, 16)] = _Z0 + (i % 4) * 16 + lane16

    base = b * _T
    lane = lax.iota(jnp.int32, 16)
    gdn = lax.GatherDimensionNumbers(
        offset_dims=(), collapsed_slice_dims=(0,), start_index_map=(0,))

    def shift16(c, s):
        src = jnp.maximum(lane - s, 0)
        g = lax.gather(c, src[:, None], gdn, (1,),
                       mode=lax.GatherScatterMode.PROMISE_IN_BOUNDS)
        return jnp.where(lane >= s, g, 0)

    def scan_body(j, carry):
        v = dur_v[pl.ds(j * 16, 16)]
        cum = v
        for s in (1, 2, 4, 8):
            cum = cum + shift16(cum, s)
        pos = cum - v + carry
        val = base + j * 16 + lane
        for r in range(3):
            m = (v > r) & (pos + r < _L)
            plsc.store_scatter(idx_v, [pos + r], val, mask=m)
        return carry + cum[15]

    mel_len = lax.fori_loop(0, _T // 16, scan_body, jnp.int32(0))

    prev = None
    was_zero = jnp.bool_(False)
    for c in range(512 // _CH):
        start = half * 512 + c * _CH
        valid = start < mel_len
        if prev is not None:
            prev[0].wait()
            prev[1].wait()

        @pl.when(valid)
        def _(start=start):
            pltpu.async_copy(src_hbm.at[idx_v.at[pl.ds(start, _CH)]],
                             buf_v, gsem).wait()

        @pl.when(jnp.logical_not(valid | was_zero))
        def _():
            pltpu.async_copy(src_hbm.at[pl.ds(_Z0, _CH)], buf_v, gsem).wait()

        was_zero = jnp.logical_not(valid)
        orow = b * _L + start
        prev = (
            pltpu.async_copy(buf_v.at[:, pl.ds(0, _D)],
                             text_hbm.at[pl.ds(orow, _CH)], os0),
            pltpu.async_copy(buf_v.at[:, pl.ds(_D, _D)],
                             pitch_hbm.at[pl.ds(orow, _CH)], os1),
        )
    prev[0].wait()
    prev[1].wait()


def _length_regulate(srcflat, dur):
    return pl.kernel(
        _lr_body,
        out_type=[
            jax.ShapeDtypeStruct((_B * _L, _D), jnp.float32),
            jax.ShapeDtypeStruct((_B * _L, _D), jnp.float32),
        ],
        mesh=plsc.VectorSubcoreMesh(core_axis_name="c", subcore_axis_name="s",
                                    num_cores=_NC, num_subcores=_NS),
        compiler_params=pltpu.CompilerParams(needs_layout_passes=False),
        scratch_types=[
            pltpu.VMEM((_T,), jnp.int32),
            pltpu.VMEM((_L,), jnp.int32),
            pltpu.VMEM((_CH, 2 * _D), jnp.float32),
        ] + [pltpu.SemaphoreType.DMA] * 3,
    )(srcflat, dur)


def kernel(x, speaker_embedding, src_mask, mel_mask, max_len, pitch_target,
           duration_target,
           w_dc1, b_dc1, g_d1, be_d1, w_dc2, b_dc2, g_d2, be_d2, w_dlin, b_dlin,
           w_pc1, b_pc1, g_p1, be_p1, w_pc2, b_pc2, g_p2, be_p2, w_plin, b_plin,
           w_pe, b_pe):
    taps = lambda w: jnp.transpose(w, (2, 1, 0))
    ld, pp, pe = _front(
        x, speaker_embedding, pitch_target[:, :, None],
        taps(w_dc1), b_dc1, g_d1, be_d1, taps(w_dc2), b_dc2, g_d2, be_d2,
        w_dlin,
        taps(w_pc1), b_pc1, g_p1, be_p1, taps(w_pc2), b_pc2, g_p2, be_p2,
        w_plin,
        jnp.transpose(w_pe[:, 0, :]), b_pe)

    text_f, pitch_f = _length_regulate(
        pe.reshape((_B + 1) * _T, 2 * _D), duration_target)

    log_dur = jnp.where(src_mask, 0.0, ld[:, 0] + b_dlin)
    pitch_pred = jnp.where(src_mask, 0.0, pp[:, 0] + b_plin)
    mel_len = jnp.sum(duration_target, axis=1)
    return (text_f.reshape(_B, _L, _D), pitch_f.reshape(_B, _L, _D),
            pitch_pred, log_dur, duration_target, mel_len, mel_mask)


# speedup vs baseline: 3.7408x; 1.0718x over previous
import functools

import jax
import jax.numpy as jnp
from jax import lax
from jax.experimental import pallas as pl
from jax.experimental.pallas import tpu as pltpu
from jax.experimental.pallas import tpu_sc as plsc

_B, _T, _D, _L = 16, 512, 512, 1024
_NC, _NS = 2, 16
_CH = 64
_Z0 = _B * _T


def _ln(h, g, b):
    mu = jnp.mean(h, axis=-1, keepdims=True)
    var = jnp.mean((h - mu) * (h - mu), axis=-1, keepdims=True)
    return (h - mu) * lax.rsqrt(var + 1e-5) * g + b


def _front_body(x_ref, spk_ref, pt_ref,
                w1_ref, bd1_ref, gd1_ref, bed1_ref,
                wd2_ref, bd2_ref, gd2_ref, bed2_ref, wdl_ref,
                bp1_ref, gp1_ref, bep1_ref,
                wp2_ref, bp2_ref, gp2_ref, bep2_ref, wpl_ref,
                wpe_ref, bpe_ref,
                ld_ref, pp_ref, pe_ref):
    bi = pl.program_id(0)

    @pl.when(bi == _B)
    def _():
        pe_ref[0] = jnp.zeros((_T, 2 * _D), jnp.float32)

    @pl.when(bi < _B)
    def _():
        _front_compute(x_ref, spk_ref, pt_ref,
                       w1_ref, bd1_ref, gd1_ref, bed1_ref,
                       wd2_ref, bd2_ref, gd2_ref, bed2_ref, wdl_ref,
                       bp1_ref, gp1_ref, bep1_ref,
                       wp2_ref, bp2_ref, gp2_ref, bep2_ref, wpl_ref,
                       wpe_ref, bpe_ref, ld_ref, pp_ref, pe_ref)


def _front_compute(x_ref, spk_ref, pt_ref,
                   w1_ref, bd1_ref, gd1_ref, bed1_ref,
                   wd2_ref, bd2_ref, gd2_ref, bed2_ref, wdl_ref,
                   bp1_ref, gp1_ref, bep1_ref,
                   wp2_ref, bp2_ref, gp2_ref, bep2_ref, wpl_ref,
                   wpe_ref, bpe_ref, ld_ref, pp_ref, pe_ref):
    xb = x_ref[0]
    zrow = jnp.zeros((1, _D), jnp.float32)

    def win3(m):
        return jnp.concatenate(
            [jnp.concatenate([zrow, m[:-1]], 0), m,
             jnp.concatenate([m[1:], zrow], 0)], 1)

    h = win3(xb) @ w1_ref[...]
    hd = _ln(jnp.maximum(h[:, :_D] + bd1_ref[...], 0.0),
             gd1_ref[...], bed1_ref[...])
    hp = _ln(jnp.maximum(h[:, _D:] + bp1_ref[...], 0.0),
             gp1_ref[...], bep1_ref[...])

    def tail(h1, w2, b2, g2, be2, wl):
        h2 = win3(h1) @ w2 + b2
        h2 = _ln(jnp.maximum(h2, 0.0), g2, be2)
        return jnp.sum(h2 * wl, axis=-1)

    ld_ref[0, 0, :] = tail(hd, wd2_ref[...], bd2_ref[...], gd2_ref[...],
                           bed2_ref[...], wdl_ref[...])
    pp_ref[0, 0, :] = tail(hp, wp2_ref[...], bp2_ref[...], gp2_ref[...],
                           bep2_ref[...], wpl_ref[...])

    pt = pt_ref[0]
    rows = []
    for k in range(9):
        d = k - 4
        if d < 0:
            rows.append(jnp.concatenate(
                [jnp.zeros((1, -d), jnp.float32), pt[:, :d]], 1))
        elif d == 0:
            rows.append(pt)
        else:
            rows.append(jnp.concatenate(
                [pt[:, d:], jnp.zeros((1, d), jnp.float32)], 1))
    p9 = jnp.concatenate(rows, 0)
    acc = lax.dot_general(p9, wpe_ref[...], (((0,), (0,)), ((), ())))
    pe_ref[0, :, :_D] = xb
    pe_ref[0, :, _D:] = acc + bpe_ref[...] + spk_ref[0]


def _front(x, spk, pt3, *weights):
    clamp = lambda b: jnp.minimum(b, _B - 1)
    full = lambda a: pl.BlockSpec(a.shape, lambda b: (0,) * a.ndim)
    in_specs = [
        pl.BlockSpec((1, _T, _D), lambda b: (clamp(b), 0, 0)),
        pl.BlockSpec((1, _T, _D), lambda b: (clamp(b), 0, 0)),
        pl.BlockSpec((1, 1, _T), lambda b: (clamp(b), 0, 0)),
    ] + [full(w) for w in weights]
    return pl.pallas_call(
        _front_body,
        grid=(_B + 1,),
        in_specs=in_specs,
        out_specs=[
            pl.BlockSpec((1, 1, _T), lambda b: (clamp(b), 0, 0)),
            pl.BlockSpec((1, 1, _T), lambda b: (clamp(b), 0, 0)),
            pl.BlockSpec((1, _T, 2 * _D), lambda b: (b, 0, 0)),
        ],
        out_shape=[
            jax.ShapeDtypeStruct((_B, 1, _T), jnp.float32),
            jax.ShapeDtypeStruct((_B, 1, _T), jnp.float32),
            jax.ShapeDtypeStruct((_B + 1, _T, 2 * _D), jnp.float32),
        ],
        compiler_params=pltpu.CompilerParams(
            dimension_semantics=("arbitrary",)),
    )(x, spk, pt3, *weights)


def _lr_body(src_hbm, dur_hbm, text_hbm, pitch_hbm,
             dur_v, idx_v, buf_v, gsem, os0, os1):
    core = lax.axis_index("c")
    sub = lax.axis_index("s")
    b = core * 8 + sub // 2
    half = sub % 2

    pltpu.sync_copy(dur_hbm.at[b], dur_v)

    lane16 = lax.iota(jnp.int32, 16)
    for i in range(_L // 16):
        idx_v[pl.ds(i * 16, 16)] = _Z0 + (i % 4) * 16 + lane16

    base = b * _T
    lane = lax.iota(jnp.int32, 16)
    gdn = lax.GatherDimensionNumbers(
        offset_dims=(), collapsed_slice_dims=(0,), start_index_map=(0,))

    def shift16(c, s):
        src = jnp.maximum(lane - s, 0)
        g = lax.gather(c, src[:, None], gdn, (1,),
                       mode=lax.GatherScatterMode.PROMISE_IN_BOUNDS)
        return jnp.where(lane >= s, g, 0)

    def scan_body(j, carry):
        v = dur_v[pl.ds(j * 16, 16)]
        cum = v
        for s in (1, 2, 4, 8):
            cum = cum + shift16(cum, s)
        pos = cum - v + carry
        val = base + j * 16 + lane
        for r in range(3):
            m = (v > r) & (pos + r < _L)
            plsc.store_scatter(idx_v, [pos + r], val, mask=m)
        return carry + cum[15]

    mel_len = lax.fori_loop(0, _T // 16, scan_body, jnp.int32(0))

    prev = None
    was_zero = jnp.bool_(False)
    for c in range(512 // _CH):
        start = half * 512 + c * _CH
        valid = start < mel_len
        if prev is not None:
            prev[0].wait()
            prev[1].wait()

        @pl.when(valid)
        def _(start=start):
            pltpu.async_copy(src_hbm.at[idx_v.at[pl.ds(start, _CH)]],
                             buf_v, gsem).wait()

        @pl.when(jnp.logical_not(valid | was_zero))
        def _():
            pltpu.async_copy(src_hbm.at[pl.ds(_Z0, _CH)], buf_v, gsem).wait()

        was_zero = jnp.logical_not(valid)
        orow = b * _L + start
        prev = (
            pltpu.async_copy(buf_v.at[:, pl.ds(0, _D)],
                             text_hbm.at[pl.ds(orow, _CH)], os0),
            pltpu.async_copy(buf_v.at[:, pl.ds(_D, _D)],
                             pitch_hbm.at[pl.ds(orow, _CH)], os1),
        )
    prev[0].wait()
    prev[1].wait()


def _length_regulate(srcflat, dur):
    return pl.kernel(
        _lr_body,
        out_type=[
            jax.ShapeDtypeStruct((_B * _L, _D), jnp.float32),
            jax.ShapeDtypeStruct((_B * _L, _D), jnp.float32),
        ],
        mesh=plsc.VectorSubcoreMesh(core_axis_name="c", subcore_axis_name="s",
                                    num_cores=_NC, num_subcores=_NS),
        compiler_params=pltpu.CompilerParams(needs_layout_passes=False),
        scratch_types=[
            pltpu.VMEM((_T,), jnp.int32),
            pltpu.VMEM((_L,), jnp.int32),
            pltpu.VMEM((_CH, 2 * _D), jnp.float32),
        ] + [pltpu.SemaphoreType.DMA] * 3,
    )(srcflat, dur)


def kernel(x, speaker_embedding, src_mask, mel_mask, max_len, pitch_target,
           duration_target,
           w_dc1, b_dc1, g_d1, be_d1, w_dc2, b_dc2, g_d2, be_d2, w_dlin, b_dlin,
           w_pc1, b_pc1, g_p1, be_p1, w_pc2, b_pc2, g_p2, be_p2, w_plin, b_plin,
           w_pe, b_pe):
    t3 = lambda w: jnp.transpose(w, (2, 1, 0)).reshape(3 * _D, _D)
    w1all = jnp.concatenate([t3(w_dc1), t3(w_pc1)], axis=1)
    ld, pp, pe = _front(
        x, speaker_embedding, pitch_target[:, None, :],
        w1all, b_dc1, g_d1, be_d1, t3(w_dc2), b_dc2, g_d2, be_d2, w_dlin,
        b_pc1, g_p1, be_p1, t3(w_pc2), b_pc2, g_p2, be_p2, w_plin,
        jnp.transpose(w_pe[:, 0, :]), b_pe)

    text_f, pitch_f = _length_regulate(
        pe.reshape((_B + 1) * _T, 2 * _D), duration_target)

    log_dur = jnp.where(src_mask, 0.0, ld[:, 0] + b_dlin)
    pitch_pred = jnp.where(src_mask, 0.0, pp[:, 0] + b_plin)
    mel_len = jnp.sum(duration_target, axis=1)
    return (text_f.reshape(_B, _L, _D), pitch_f.reshape(_B, _L, _D),
            pitch_pred, log_dur, duration_target, mel_len, mel_mask)


# speedup vs baseline: 3.8571x; 1.0311x over previous
import functools

import jax
import jax.numpy as jnp
from jax import lax
from jax.experimental import pallas as pl
from jax.experimental.pallas import tpu as pltpu
from jax.experimental.pallas import tpu_sc as plsc

_B, _T, _D, _L = 16, 512, 512, 1024
_NC, _NS = 2, 16
_CH = 32
_Z0 = _B * _T


def _ln(h, g, b):
    mu = jnp.mean(h, axis=-1, keepdims=True)
    var = jnp.mean((h - mu) * (h - mu), axis=-1, keepdims=True)
    return (h - mu) * lax.rsqrt(var + 1e-5) * g + b


def _front_body(x_ref, spk_ref, pt_ref,
                w1_ref, bd1_ref, gd1_ref, bed1_ref,
                wd2_ref, bd2_ref, gd2_ref, bed2_ref, wdl_ref,
                bp1_ref, gp1_ref, bep1_ref,
                wp2_ref, bp2_ref, gp2_ref, bep2_ref, wpl_ref,
                wpe_ref, bpe_ref,
                ld_ref, pp_ref, pe_ref):
    bi = pl.program_id(0)

    @pl.when(bi == _B)
    def _():
        pe_ref[0] = jnp.zeros((_T, 2 * _D), jnp.float32)

    @pl.when(bi < _B)
    def _():
        _front_compute(x_ref, spk_ref, pt_ref,
                       w1_ref, bd1_ref, gd1_ref, bed1_ref,
                       wd2_ref, bd2_ref, gd2_ref, bed2_ref, wdl_ref,
                       bp1_ref, gp1_ref, bep1_ref,
                       wp2_ref, bp2_ref, gp2_ref, bep2_ref, wpl_ref,
                       wpe_ref, bpe_ref, ld_ref, pp_ref, pe_ref)


def _front_compute(x_ref, spk_ref, pt_ref,
                   w1_ref, bd1_ref, gd1_ref, bed1_ref,
                   wd2_ref, bd2_ref, gd2_ref, bed2_ref, wdl_ref,
                   bp1_ref, gp1_ref, bep1_ref,
                   wp2_ref, bp2_ref, gp2_ref, bep2_ref, wpl_ref,
                   wpe_ref, bpe_ref, ld_ref, pp_ref, pe_ref):
    xb = x_ref[0]
    zrow = jnp.zeros((1, _D), jnp.float32)

    def win3(m):
        return jnp.concatenate(
            [jnp.concatenate([zrow, m[:-1]], 0), m,
             jnp.concatenate([m[1:], zrow], 0)], 1)

    h = win3(xb) @ w1_ref[...]
    hd = _ln(jnp.maximum(h[:, :_D] + bd1_ref[...], 0.0),
             gd1_ref[...], bed1_ref[...])
    hp = _ln(jnp.maximum(h[:, _D:] + bp1_ref[...], 0.0),
             gp1_ref[...], bep1_ref[...])

    def tail(h1, w2, b2, g2, be2, wl):
        h2 = win3(h1) @ w2 + b2
        h2 = _ln(jnp.maximum(h2, 0.0), g2, be2)
        return jnp.sum(h2 * wl, axis=-1)

    ld_ref[0, 0, :] = tail(hd, wd2_ref[...], bd2_ref[...], gd2_ref[...],
                           bed2_ref[...], wdl_ref[...])
    pp_ref[0, 0, :] = tail(hp, wp2_ref[...], bp2_ref[...], gp2_ref[...],
                           bep2_ref[...], wpl_ref[...])

    pt = pt_ref[0]
    rows = []
    for k in range(9):
        d = k - 4
        if d < 0:
            rows.append(jnp.concatenate(
                [jnp.zeros((1, -d), jnp.float32), pt[:, :d]], 1))
        elif d == 0:
            rows.append(pt)
        else:
            rows.append(jnp.concatenate(
                [pt[:, d:], jnp.zeros((1, d), jnp.float32)], 1))
    p9 = jnp.concatenate(rows, 0)
    acc = lax.dot_general(p9, wpe_ref[...], (((0,), (0,)), ((), ())))
    pe_ref[0, :, :_D] = xb
    pe_ref[0, :, _D:] = acc + bpe_ref[...] + spk_ref[0]


def _front(x, spk, pt3, *weights):
    clamp = lambda b: jnp.minimum(b, _B - 1)
    full = lambda a: pl.BlockSpec(a.shape, lambda b: (0,) * a.ndim)
    in_specs = [
        pl.BlockSpec((1, _T, _D), lambda b: (clamp(b), 0, 0)),
        pl.BlockSpec((1, _T, _D), lambda b: (clamp(b), 0, 0)),
        pl.BlockSpec((1, 1, _T), lambda b: (clamp(b), 0, 0)),
    ] + [full(w) for w in weights]
    return pl.pallas_call(
        _front_body,
        grid=(_B + 1,),
        in_specs=in_specs,
        out_specs=[
            pl.BlockSpec((1, 1, _T), lambda b: (clamp(b), 0, 0)),
            pl.BlockSpec((1, 1, _T), lambda b: (clamp(b), 0, 0)),
            pl.BlockSpec((1, _T, 2 * _D), lambda b: (b, 0, 0)),
        ],
        out_shape=[
            jax.ShapeDtypeStruct((_B, 1, _T), jnp.float32),
            jax.ShapeDtypeStruct((_B, 1, _T), jnp.float32),
            jax.ShapeDtypeStruct((_B + 1, _T, 2 * _D), jnp.float32),
        ],
        compiler_params=pltpu.CompilerParams(
            dimension_semantics=("arbitrary",)),
    )(x, spk, pt3, *weights)


def _lr_body(src_hbm, dur_hbm, text_hbm, pitch_hbm,
             dur_v, idx_v, buf0_v, buf1_v, gs0, gs1, os0, os1, os2, os3):
    core = lax.axis_index("c")
    sub = lax.axis_index("s")
    b = core * 8 + sub // 2
    half = sub % 2

    pltpu.sync_copy(dur_hbm.at[b], dur_v)

    lane16 = lax.iota(jnp.int32, 16)
    for i in range(_L // 16):
        idx_v[pl.ds(i * 16, 16)] = _Z0 + (i % 4) * 16 + lane16

    base = b * _T
    lane = lax.iota(jnp.int32, 16)
    gdn = lax.GatherDimensionNumbers(
        offset_dims=(), collapsed_slice_dims=(0,), start_index_map=(0,))

    def shift16(c, s):
        src = jnp.maximum(lane - s, 0)
        g = lax.gather(c, src[:, None], gdn, (1,),
                       mode=lax.GatherScatterMode.PROMISE_IN_BOUNDS)
        return jnp.where(lane >= s, g, 0)

    def scan_body(j, carry):
        v = dur_v[pl.ds(j * 16, 16)]
        cum = v
        for s in (1, 2, 4, 8):
            cum = cum + shift16(cum, s)
        pos = cum - v + carry
        val = base + j * 16 + lane
        for r in range(3):
            m = (v > r) & (pos + r < _L)
            plsc.store_scatter(idx_v, [pos + r], val, mask=m)
        return carry + cum[15]

    mel_len = lax.fori_loop(0, _T // 16, scan_body, jnp.int32(0))

    n_ch = 512 // _CH
    bufs = (buf0_v, buf1_v)
    gsems = (gs0, gs1)
    osems = ((os0, os1), (os2, os3))
    was_zero = [jnp.bool_(False), jnp.bool_(False)]
    issued = [None] * n_ch
    wbs = [None] * n_ch

    def fill(c):
        i = c % 2
        start = half * 512 + c * _CH
        valid = start < mel_len
        do_zero = jnp.logical_not(valid | was_zero[i])

        @pl.when(valid)
        def _(start=start, i=i):
            pltpu.async_copy(src_hbm.at[idx_v.at[pl.ds(start, _CH)]],
                             bufs[i], gsems[i])

        @pl.when(do_zero)
        def _(i=i):
            pltpu.async_copy(src_hbm.at[pl.ds(_Z0, _CH)], bufs[i], gsems[i])

        was_zero[i] = jnp.logical_not(valid)
        return valid | do_zero

    issued[0] = fill(0)
    for c in range(n_ch):
        i = c % 2
        if c + 1 < n_ch:
            if c - 1 >= 0:
                wbs[c - 1][0].wait()
                wbs[c - 1][1].wait()
            issued[c + 1] = fill(c + 1)

        @pl.when(issued[c])
        def _(i=i):
            pltpu.make_async_copy(src_hbm.at[pl.ds(_Z0, _CH)],
                                  bufs[i], gsems[i]).wait()

        orow = b * _L + half * 512 + c * _CH
        wbs[c] = (
            pltpu.async_copy(bufs[i].at[:, pl.ds(0, _D)],
                             text_hbm.at[pl.ds(orow, _CH)], osems[i][0]),
            pltpu.async_copy(bufs[i].at[:, pl.ds(_D, _D)],
                             pitch_hbm.at[pl.ds(orow, _CH)], osems[i][1]),
        )
    for c in (n_ch - 2, n_ch - 1):
        wbs[c][0].wait()
        wbs[c][1].wait()


def _length_regulate(srcflat, dur):
    return pl.kernel(
        _lr_body,
        out_type=[
            jax.ShapeDtypeStruct((_B * _L, _D), jnp.float32),
            jax.ShapeDtypeStruct((_B * _L, _D), jnp.float32),
        ],
        mesh=plsc.VectorSubcoreMesh(core_axis_name="c", subcore_axis_name="s",
                                    num_cores=_NC, num_subcores=_NS),
        compiler_params=pltpu.CompilerParams(needs_layout_passes=False),
        scratch_types=[
            pltpu.VMEM((_T,), jnp.int32),
            pltpu.VMEM((_L,), jnp.int32),
            pltpu.VMEM((_CH, 2 * _D), jnp.float32),
            pltpu.VMEM((_CH, 2 * _D), jnp.float32),
        ] + [pltpu.SemaphoreType.DMA] * 6,
    )(srcflat, dur)


def kernel(x, speaker_embedding, src_mask, mel_mask, max_len, pitch_target,
           duration_target,
           w_dc1, b_dc1, g_d1, be_d1, w_dc2, b_dc2, g_d2, be_d2, w_dlin, b_dlin,
           w_pc1, b_pc1, g_p1, be_p1, w_pc2, b_pc2, g_p2, be_p2, w_plin, b_plin,
           w_pe, b_pe):
    t3 = lambda w: jnp.transpose(w, (2, 1, 0)).reshape(3 * _D, _D)
    w1all = jnp.concatenate([t3(w_dc1), t3(w_pc1)], axis=1)
    ld, pp, pe = _front(
        x, speaker_embedding, pitch_target[:, None, :],
        w1all, b_dc1, g_d1, be_d1, t3(w_dc2), b_dc2, g_d2, be_d2, w_dlin,
        b_pc1, g_p1, be_p1, t3(w_pc2), b_pc2, g_p2, be_p2, w_plin,
        jnp.transpose(w_pe[:, 0, :]), b_pe)

    text_f, pitch_f = _length_regulate(
        pe.reshape((_B + 1) * _T, 2 * _D), duration_target)

    log_dur = jnp.where(src_mask, 0.0, ld[:, 0] + b_dlin)
    pitch_pred = jnp.where(src_mask, 0.0, pp[:, 0] + b_plin)
    mel_len = jnp.sum(duration_target, axis=1)
    return (text_f.reshape(_B, _L, _D), pitch_f.reshape(_B, _L, _D),
            pitch_pred, log_dur, duration_target, mel_len, mel_mask)
